# Initial kernel scaffold; baseline (speedup 1.0000x reference)
import jax
import jax.numpy as jnp
from jax.experimental import pallas as pl


def kernel(source, emb_lut):
    raise NotImplementedError("write your pallas kernel here")


# speedup vs baseline: 6.3503x; 6.3503x over previous
import functools

import jax
import jax.numpy as jnp
from jax import lax
from jax.experimental import pallas as pl
from jax.experimental.pallas import tpu as pltpu
from jax.experimental.pallas import tpu_sc as plsc

EMB = 128
N_ROWS = 4096 * 200
NC, NS = 2, 16
NW = NC * NS
ROWS_PER_W = N_ROWS // NW
CHUNK = 128
NCHUNK = ROWS_PER_W // CHUNK


def _make_lookup():
  mesh = plsc.VectorSubcoreMesh(core_axis_name="c", subcore_axis_name="s")

  @functools.partial(
      pl.kernel,
      mesh=mesh,
      out_type=jax.ShapeDtypeStruct((N_ROWS, EMB), jnp.float32),
      scratch_types=[
          pltpu.VMEM((NCHUNK, CHUNK), jnp.int32),
          pltpu.VMEM((CHUNK, EMB), jnp.float32),
          pltpu.SemaphoreType.DMA,
      ],
  )
  def lookup(table_hbm, idx_hbm, out_hbm, idx_v, rows_v, sem):
    wid = lax.axis_index("s") * NC + lax.axis_index("c")
    base = wid * ROWS_PER_W
    pltpu.sync_copy(idx_hbm.at[wid], idx_v)

    def step(j, carry):
      pltpu.async_copy(table_hbm.at[idx_v.at[j]], rows_v, sem).wait()
      pltpu.sync_copy(rows_v, out_hbm.at[pl.ds(base + j * CHUNK, CHUNK)])
      return carry

    lax.fori_loop(0, NCHUNK, step, 0)

  return lookup


_lookup = _make_lookup()


def kernel(source, emb_lut):
  idx = source.reshape(NW, NCHUNK, CHUNK).astype(jnp.int32)
  out = _lookup(emb_lut, idx)
  return out.reshape(source.shape[0], source.shape[1], EMB)


# speedup vs baseline: 9.2285x; 1.4532x over previous
import functools

import jax
import jax.numpy as jnp
from jax import lax
from jax.experimental import pallas as pl
from jax.experimental.pallas import tpu as pltpu
from jax.experimental.pallas import tpu_sc as plsc

EMB = 128
N_ROWS = 4096 * 200
NC, NS = 2, 16
NW = NC * NS
ROWS_PER_W = N_ROWS // NW
CHUNK = 128
NCHUNK = ROWS_PER_W // CHUNK
NBUF = 4
NGROUP = NCHUNK // NBUF


def _make_lookup():
  mesh = plsc.VectorSubcoreMesh(core_axis_name="c", subcore_axis_name="s")

  @functools.partial(
      pl.kernel,
      mesh=mesh,
      out_type=jax.ShapeDtypeStruct((N_ROWS, EMB), jnp.float32),
      scratch_types=[
          pltpu.VMEM((NCHUNK, CHUNK), jnp.int32),
          pltpu.VMEM((NBUF, CHUNK, EMB), jnp.float32),
      ]
      + [pltpu.SemaphoreType.DMA] * (2 * NBUF),
  )
  def lookup(table_hbm, idx_hbm, out_hbm, idx_v, bufs, *sems):
    gsem = sems[:NBUF]
    osem = sems[NBUF:]
    wid = lax.axis_index("s") * NC + lax.axis_index("c")
    base = wid * ROWS_PER_W
    pltpu.sync_copy(idx_hbm.at[wid], idx_v)

    def start_gather(j, b):
      pltpu.async_copy(table_hbm.at[idx_v.at[j]], bufs.at[b], gsem[b])

    def wait_gather(b):
      pltpu.make_async_copy(
          table_hbm.at[pl.ds(0, CHUNK)], bufs.at[b], gsem[b]).wait()

    def start_out(j, b):
      pltpu.async_copy(
          bufs.at[b], out_hbm.at[pl.ds(base + j * CHUNK, CHUNK)], osem[b])

    def wait_out(b):
      pltpu.make_async_copy(
          bufs.at[b], out_hbm.at[pl.ds(0, CHUNK)], osem[b]).wait()

    for b in range(NBUF):
      start_gather(b, b)

    def group(g, carry):
      for b in range(NBUF):
        j = g * NBUF + b
        wait_gather(b)
        start_out(j, b)
        wait_out(b)
        start_gather(j + NBUF, b)
      return carry

    lax.fori_loop(0, NGROUP - 1, group, 0)

    for b in range(NBUF):
      wait_gather(b)
      start_out((NGROUP - 1) * NBUF + b, b)
    for b in range(NBUF):
      wait_out(b)

  return lookup


_lookup = _make_lookup()


def kernel(source, emb_lut):
  idx = source.reshape(NW, NCHUNK, CHUNK).astype(jnp.int32)
  out = _lookup(emb_lut, idx)
  return out.reshape(source.shape[0], source.shape[1], EMB)


# speedup vs baseline: 9.2601x; 1.0034x over previous
import functools

import jax
import jax.numpy as jnp
from jax import lax
from jax.experimental import pallas as pl
from jax.experimental.pallas import tpu as pltpu
from jax.experimental.pallas import tpu_sc as plsc

EMB = 128
N_ROWS = 4096 * 200
NC, NS = 2, 16
NW = NC * NS
ROWS_PER_W = N_ROWS // NW
CHUNK = 128
NCHUNK = ROWS_PER_W // CHUNK
NBUF = 5
DEFER = 2
NGROUP = NCHUNK // NBUF


def _make_lookup():
  mesh = plsc.VectorSubcoreMesh(core_axis_name="c", subcore_axis_name="s")

  @functools.partial(
      pl.kernel,
      mesh=mesh,
      out_type=jax.ShapeDtypeStruct((N_ROWS, EMB), jnp.float32),
      scratch_types=[
          pltpu.VMEM((NCHUNK, CHUNK), jnp.int32),
          pltpu.VMEM((NBUF, CHUNK, EMB), jnp.float32),
      ]
      + [pltpu.SemaphoreType.DMA] * (2 * NBUF),
  )
  def lookup(table_hbm, idx_hbm, out_hbm, idx_v, bufs, *sems):
    gsem = sems[:NBUF]
    osem = sems[NBUF:]
    wid = lax.axis_index("s") * NC + lax.axis_index("c")
    base = wid * ROWS_PER_W
    pltpu.sync_copy(idx_hbm.at[wid], idx_v)

    def start_gather(j, b):
      pltpu.async_copy(table_hbm.at[idx_v.at[j]], bufs.at[b], gsem[b])

    def wait_gather(b):
      pltpu.make_async_copy(
          table_hbm.at[pl.ds(0, CHUNK)], bufs.at[b], gsem[b]).wait()

    def start_out(j, b):
      pltpu.async_copy(
          bufs.at[b], out_hbm.at[pl.ds(base + j * CHUNK, CHUNK)], osem[b])

    def wait_out(b):
      pltpu.make_async_copy(
          bufs.at[b], out_hbm.at[pl.ds(0, CHUNK)], osem[b]).wait()

    def step(j, b, recycle, restart):
      wait_gather(b)
      start_out(j, b)
      b2 = (b - DEFER) % NBUF
      if recycle:
        wait_out(b2)
      if restart:
        start_gather(j + NBUF - DEFER, b2)

    for b in range(NBUF):
      start_gather(b, b)

    for b in range(NBUF):
      step(b, b, recycle=(b >= DEFER), restart=(b >= DEFER))

    def group(g, carry):
      for b in range(NBUF):
        step(g * NBUF + b, b, recycle=True, restart=True)
      return carry

    lax.fori_loop(1, NGROUP - 1, group, 0)

    for b in range(NBUF):
      j = (NGROUP - 1) * NBUF + b
      step(j, b, recycle=True, restart=(j + NBUF - DEFER < NCHUNK))

    for b in range(NBUF - DEFER, NBUF):
      wait_out(b)

  return lookup


_lookup = _make_lookup()


def kernel(source, emb_lut):
  idx = source.reshape(NW, NCHUNK, CHUNK).astype(jnp.int32)
  out = _lookup(emb_lut, idx)
  return out.reshape(source.shape[0], source.shape[1], EMB)
